# baseline (speedup 1.0000x reference)
import functools

import jax
import jax.numpy as jnp
from jax import lax
from jax.experimental import pallas as pl
from jax.experimental.pallas import tpu as pltpu
from jax.experimental.pallas import tpu_sc as plsc

POS_B = 4096
NEG_B = 16384
DIM = 128
NEG_RATIO = NEG_B // POS_B
N_RULES = 20
N_RULE_PAD = 32
MARGIN = 1.0
RULE_WEIGHT = 0.5

ENT_N = 2 * POS_B + 2 * NEG_B
REL_N = POS_B + NEG_B

_NW = 32
_CH = 128
_ENT_PW = ENT_N // _NW
_REL_PW = REL_N // _NW
_ENT_CH = _ENT_PW // _CH
_REL_CH = _REL_PW // _CH

_NBUF = 7

_PB = 1024
_GRID = POS_B // _PB
_S = POS_B // _PB


_sc_mesh = plsc.VectorSubcoreMesh(core_axis_name="c", subcore_axis_name="s")


@functools.partial(
    pl.kernel,
    mesh=_sc_mesh,
    out_type=(
        jax.ShapeDtypeStruct((ENT_N, DIM), jnp.float32),
        jax.ShapeDtypeStruct((REL_N, DIM), jnp.float32),
        jax.ShapeDtypeStruct((REL_N, DIM), jnp.float32),
        jax.ShapeDtypeStruct((N_RULE_PAD, DIM), jnp.float32),
        jax.ShapeDtypeStruct((N_RULE_PAD, DIM), jnp.float32),
    ),
    scratch_types=[
        pltpu.VMEM((_ENT_PW,), jnp.int32),
        pltpu.VMEM((_REL_PW,), jnp.int32),
        pltpu.VMEM((N_RULE_PAD,), jnp.int32),
        pltpu.VMEM((_NBUF, _CH, DIM), jnp.float32),
        pltpu.VMEM((N_RULE_PAD, DIM), jnp.float32),
        pltpu.SemaphoreType.DMA,
    ] + [pltpu.SemaphoreType.DMA] * (2 * _NBUF),
)
def _sc_gather(ent_hbm, rel_hbm, norm_hbm, eidx_hbm, ridx_hbm, rulidx_hbm,
               out_ent, out_rel, out_norm, out_dr, out_wr,
               idx_e, idx_r, idx_rul, rows, rows_rul, sem, *ring_sems):
    wid = lax.axis_index("s") * 2 + lax.axis_index("c")
    gsem, ssem = ring_sems[:_NBUF], ring_sems[_NBUF:]

    pltpu.sync_copy(eidx_hbm.at[pl.ds(wid * _ENT_PW, _ENT_PW)], idx_e)
    pltpu.sync_copy(ridx_hbm.at[pl.ds(wid * _REL_PW, _REL_PW)], idx_r)

    items = []
    for c in range(_ENT_CH):
        items.append((ent_hbm, idx_e.at[pl.ds(c * _CH, _CH)],
                      out_ent, wid * _ENT_PW + c * _CH))
    for c in range(_REL_CH):
        isl = idx_r.at[pl.ds(c * _CH, _CH)]
        off = wid * _REL_PW + c * _CH
        items.append((rel_hbm, isl, out_rel, off))
        items.append((norm_hbm, isl, out_norm, off))

    n = len(items)
    gh = [None] * _NBUF
    sh = [None] * _NBUF
    issued = 0
    for k in range(n):
        while issued < min(n, k + _NBUF):
            b = issued % _NBUF
            if sh[b] is not None:
                sh[b].wait()
            tbl, isl, _, _ = items[issued]
            gh[b] = pltpu.async_copy(tbl.at[isl], rows.at[b], gsem[b])
            issued += 1
        b = k % _NBUF
        gh[b].wait()
        _, _, dst, off = items[k]
        sh[b] = pltpu.async_copy(rows.at[b], dst.at[pl.ds(off, _CH)], ssem[b])
    for b in range(min(_NBUF, n)):
        sh[b].wait()

    @pl.when(wid == 0)
    def _():
        pltpu.sync_copy(rulidx_hbm, idx_rul)
        pltpu.async_copy(rel_hbm.at[idx_rul], rows_rul, sem).wait()
        pltpu.sync_copy(rows_rul, out_dr)
        pltpu.async_copy(norm_hbm.at[idx_rul], rows_rul, sem).wait()
        pltpu.sync_copy(rows_rul, out_wr)


def _normw(w):
    return w / (jnp.sqrt(jnp.sum(w * w, axis=-1, keepdims=True)) + 1e-9)


def _tc_body(hp, tp, hn0, hn1, hn2, hn3, tn0, tn1, tn2, tn3,
             dp, dn0, dn1, dn2, dn3, wp, wn0, wn1, wn2, wn3,
             dr, wr, posr, r1b, confb, out):
    i = pl.program_id(0)

    def _score_u(u, d, w):
        wn = _normw(w)
        al = jnp.sum(wn * u, axis=-1, keepdims=True)
        v = u - al * wn + d
        return -jnp.sqrt(jnp.sum(v * v, axis=-1, keepdims=True) + 1e-12)

    up = hp[...] - tp[...]
    ps = _score_u(up, dp[...], wp[...])

    basic = jnp.float32(0.0)
    for hn, tn, dn, wn in ((hn0, tn0, dn0, wn0), (hn1, tn1, dn1, wn1),
                           (hn2, tn2, dn2, wn2), (hn3, tn3, dn3, wn3)):
        ns = _score_u(hn[...] - tn[...], dn[...], wn[...])
        basic = basic + jnp.sum(jax.nn.relu(MARGIN - ps + ns))

    drv = dr[...]
    wrv = _normw(wr[...])
    dn_ = (((1,), (1,)), ((), ()))
    alr = lax.dot_general(up, wrv, dn_, preferred_element_type=jnp.float32)
    ber = lax.dot_general(up, drv, dn_, preferred_element_type=jnp.float32)
    ones = jnp.ones((1, DIM), jnp.float32)
    ddr = lax.dot_general(ones, drv * drv, dn_, preferred_element_type=jnp.float32)
    wdr = lax.dot_general(ones, wrv * drv, dn_, preferred_element_type=jnp.float32)
    nu = jnp.sum(up * up, axis=-1, keepdims=True)
    dist2 = nu - alr * alr + ddr + 2.0 * ber - 2.0 * alr * wdr
    rsc = -jnp.sqrt(jnp.maximum(dist2, 0.0) + 1e-12)
    mask = posr[...] == r1b[0:1, :]
    rulep = -jnp.sum(jnp.where(mask, confb[0:1, :] * rsc, 0.0))

    part = basic * (1.0 / NEG_B) + RULE_WEIGHT * rulep

    @pl.when(i == 0)
    def _():
        out[...] = jnp.zeros_like(out)

    out[...] += part


def _tc_call(ent_rows, rel_rows, norm_rows, dr_rows, wr_rows, posr, r1b, confb):
    ebs = lambda f: pl.BlockSpec((_PB, DIM), f)
    specs = []
    specs.append(ebs(lambda i: (i, 0)))
    specs.append(ebs(lambda i: (i + _S, 0)))
    for q in range(NEG_RATIO):
        specs.append(ebs(lambda i, q=q: (2 * _S + _S * q + i, 0)))
    for q in range(NEG_RATIO):
        specs.append(ebs(lambda i, q=q: (6 * _S + _S * q + i, 0)))
    specs.append(ebs(lambda i: (i, 0)))
    for q in range(NEG_RATIO):
        specs.append(ebs(lambda i, q=q: (_S + _S * q + i, 0)))
    specs.append(ebs(lambda i: (i, 0)))
    for q in range(NEG_RATIO):
        specs.append(ebs(lambda i, q=q: (_S + _S * q + i, 0)))
    specs.append(pl.BlockSpec((N_RULE_PAD, DIM), lambda i: (0, 0)))
    specs.append(pl.BlockSpec((N_RULE_PAD, DIM), lambda i: (0, 0)))
    specs.append(pl.BlockSpec((_PB, 1), lambda i: (i, 0)))
    specs.append(pl.BlockSpec((8, N_RULE_PAD), lambda i: (0, 0)))
    specs.append(pl.BlockSpec((8, N_RULE_PAD), lambda i: (0, 0)))
    return pl.pallas_call(
        _tc_body,
        grid=(_GRID,),
        in_specs=specs,
        out_specs=pl.BlockSpec((1, 1), lambda i: (0, 0)),
        out_shape=jax.ShapeDtypeStruct((1, 1), jnp.float32),
    )(ent_rows, ent_rows, ent_rows, ent_rows, ent_rows, ent_rows,
      ent_rows, ent_rows, ent_rows, ent_rows,
      rel_rows, rel_rows, rel_rows, rel_rows, rel_rows,
      norm_rows, norm_rows, norm_rows, norm_rows, norm_rows,
      dr_rows, wr_rows, posr, r1b, confb)


def kernel(pos_triples, neg_triples, ent_emb, rel_emb, norm_vec,
           rule_r1, rule_r2, rule_conf):
    ph, pr, pt = pos_triples[:, 0], pos_triples[:, 1], pos_triples[:, 2]
    nh, nr, nt = neg_triples[:, 0], neg_triples[:, 1], neg_triples[:, 2]

    qmaj = lambda x: x.reshape(POS_B, NEG_RATIO).T.reshape(-1)
    nhq, ntq, nrq = qmaj(nh), qmaj(nt), qmaj(nr)

    eidx = jnp.concatenate([ph, pt, nhq, ntq])
    ridx = jnp.concatenate([pr, nrq])
    rulidx = jnp.concatenate(
        [rule_r2, jnp.zeros((N_RULE_PAD - N_RULES,), jnp.int32)])

    ent_rows, rel_rows, norm_rows, dr_rows, wr_rows = _sc_gather(
        ent_emb, rel_emb, norm_vec, eidx, ridx, rulidx)

    posr = pr.reshape(POS_B, 1)
    pad_i = jnp.full((N_RULE_PAD - N_RULES,), -1, jnp.int32)
    r1b = jnp.broadcast_to(
        jnp.concatenate([rule_r1, pad_i])[None, :], (8, N_RULE_PAD))
    confb = jnp.broadcast_to(
        jnp.concatenate([rule_conf, jnp.zeros((N_RULE_PAD - N_RULES,),
                                              jnp.float32)])[None, :],
        (8, N_RULE_PAD))

    loss = _tc_call(ent_rows, rel_rows, norm_rows, dr_rows, wr_rows,
                    posr, r1b, confb)
    return loss.reshape(())


# speedup vs baseline: 1.0775x; 1.0775x over previous
import functools

import jax
import jax.numpy as jnp
from jax import lax
from jax.experimental import pallas as pl
from jax.experimental.pallas import tpu as pltpu
from jax.experimental.pallas import tpu_sc as plsc

POS_B = 4096
NEG_B = 16384
DIM = 128
NUM_REL = 1000
NEG_RATIO = NEG_B // POS_B
N_RULES = 20
N_RULE_PAD = 32
MARGIN = 1.0
RULE_WEIGHT = 0.5

ENT_N = 2 * POS_B + 2 * NEG_B

_NW = 32
_CH = 128
_ENT_PW = ENT_N // _NW
_ENT_CH = _ENT_PW // _CH

_NBUF = 6

_PB = 1024
_GRID = POS_B // _PB
_S = POS_B // _PB


_sc_mesh = plsc.VectorSubcoreMesh(core_axis_name="c", subcore_axis_name="s")


@functools.partial(
    pl.kernel,
    mesh=_sc_mesh,
    out_type=(
        jax.ShapeDtypeStruct((ENT_N, DIM), jnp.float32),
        jax.ShapeDtypeStruct((N_RULE_PAD, DIM), jnp.float32),
        jax.ShapeDtypeStruct((N_RULE_PAD, DIM), jnp.float32),
    ),
    scratch_types=[
        pltpu.VMEM((_ENT_PW,), jnp.int32),
        pltpu.VMEM((N_RULE_PAD,), jnp.int32),
        pltpu.VMEM((_NBUF, _CH, DIM), jnp.float32),
        pltpu.VMEM((N_RULE_PAD, DIM), jnp.float32),
        pltpu.SemaphoreType.DMA,
    ] + [pltpu.SemaphoreType.DMA] * (2 * _NBUF),
)
def _sc_gather(ent_hbm, rel_hbm, norm_hbm, eidx_hbm, rulidx_hbm,
               out_ent, out_dr, out_wr,
               idx_e, idx_rul, rows, rows_rul, sem, *ring_sems):
    wid = lax.axis_index("s") * 2 + lax.axis_index("c")
    gsem, ssem = ring_sems[:_NBUF], ring_sems[_NBUF:]

    pltpu.sync_copy(eidx_hbm.at[pl.ds(wid * _ENT_PW, _ENT_PW)], idx_e)

    n = _ENT_CH
    gh = [None] * _NBUF
    sh = [None] * _NBUF
    issued = 0
    for k in range(n):
        while issued < min(n, k + _NBUF):
            b = issued % _NBUF
            if sh[b] is not None:
                sh[b].wait()
            gh[b] = pltpu.async_copy(
                ent_hbm.at[idx_e.at[pl.ds(issued * _CH, _CH)]],
                rows.at[b], gsem[b])
            issued += 1
        b = k % _NBUF
        gh[b].wait()
        sh[b] = pltpu.async_copy(
            rows.at[b], out_ent.at[pl.ds(wid * _ENT_PW + k * _CH, _CH)],
            ssem[b])
    for b in range(min(_NBUF, n)):
        sh[b].wait()

    @pl.when(wid == 0)
    def _():
        pltpu.sync_copy(rulidx_hbm, idx_rul)
        pltpu.async_copy(rel_hbm.at[idx_rul], rows_rul, sem).wait()
        pltpu.sync_copy(rows_rul, out_dr)
        pltpu.async_copy(norm_hbm.at[idx_rul], rows_rul, sem).wait()
        pltpu.sync_copy(rows_rul, out_wr)


def _normw(w):
    return w / (jnp.sqrt(jnp.sum(w * w, axis=-1, keepdims=True)) + 1e-9)


def _tc_body(hp, tp, hn0, hn1, hn2, hn3, tn0, tn1, tn2, tn3,
             relt, normt, posr, nr0, nr1, nr2, nr3,
             dr, wr, r1b, confb, out):
    i = pl.program_id(0)

    relb = relt[...].astype(jnp.bfloat16)
    normb = normt[...].astype(jnp.bfloat16)
    iot = lax.broadcasted_iota(jnp.int32, (1, NUM_REL), 1)
    dsel_n = (((1,), (0,)), ((), ()))

    def _rel_rows(ridx):
        oh = (ridx[...] == iot).astype(jnp.bfloat16)
        d = lax.dot_general(oh, relb, dsel_n,
                            preferred_element_type=jnp.float32)
        w = lax.dot_general(oh, normb, dsel_n,
                            preferred_element_type=jnp.float32)
        return d, w

    def _score_u(u, d, w):
        wn = _normw(w)
        al = jnp.sum(wn * u, axis=-1, keepdims=True)
        v = u - al * wn + d
        return -jnp.sqrt(jnp.sum(v * v, axis=-1, keepdims=True) + 1e-12)

    up = hp[...] - tp[...]
    dp, wp = _rel_rows(posr)
    ps = _score_u(up, dp, wp)

    basic = jnp.float32(0.0)
    for hn, tn, rn in ((hn0, tn0, nr0), (hn1, tn1, nr1),
                       (hn2, tn2, nr2), (hn3, tn3, nr3)):
        dn, wn_ = _rel_rows(rn)
        ns = _score_u(hn[...] - tn[...], dn, wn_)
        basic = basic + jnp.sum(jax.nn.relu(MARGIN - ps + ns))

    drv = dr[...]
    wrv = _normw(wr[...])
    dn_ = (((1,), (1,)), ((), ()))
    alr = lax.dot_general(up, wrv, dn_, preferred_element_type=jnp.float32)
    ber = lax.dot_general(up, drv, dn_, preferred_element_type=jnp.float32)
    ones = jnp.ones((1, DIM), jnp.float32)
    ddr = lax.dot_general(ones, drv * drv, dn_,
                          preferred_element_type=jnp.float32)
    wdr = lax.dot_general(ones, wrv * drv, dn_,
                          preferred_element_type=jnp.float32)
    nu = jnp.sum(up * up, axis=-1, keepdims=True)
    dist2 = nu - alr * alr + ddr + 2.0 * ber - 2.0 * alr * wdr
    rsc = -jnp.sqrt(jnp.maximum(dist2, 0.0) + 1e-12)
    mask = posr[...] == r1b[0:1, :]
    rulep = -jnp.sum(jnp.where(mask, confb[0:1, :] * rsc, 0.0))

    part = basic * (1.0 / NEG_B) + RULE_WEIGHT * rulep

    @pl.when(i == 0)
    def _():
        out[...] = jnp.zeros_like(out)

    out[...] += part


def _tc_call(ent_rows, rel_emb, norm_vec, posr, nrq, dr_rows, wr_rows,
             r1b, confb):
    ebs = lambda f: pl.BlockSpec((_PB, DIM), f)
    ibs = lambda f: pl.BlockSpec((_PB, 1), f)
    specs = []
    specs.append(ebs(lambda i: (i, 0)))
    specs.append(ebs(lambda i: (i + _S, 0)))
    for q in range(NEG_RATIO):
        specs.append(ebs(lambda i, q=q: (2 * _S + _S * q + i, 0)))
    for q in range(NEG_RATIO):
        specs.append(ebs(lambda i, q=q: (6 * _S + _S * q + i, 0)))
    specs.append(pl.BlockSpec((NUM_REL, DIM), lambda i: (0, 0)))
    specs.append(pl.BlockSpec((NUM_REL, DIM), lambda i: (0, 0)))
    specs.append(ibs(lambda i: (i, 0)))
    for q in range(NEG_RATIO):
        specs.append(ibs(lambda i, q=q: (_S * q + i, 0)))
    specs.append(pl.BlockSpec((N_RULE_PAD, DIM), lambda i: (0, 0)))
    specs.append(pl.BlockSpec((N_RULE_PAD, DIM), lambda i: (0, 0)))
    specs.append(pl.BlockSpec((8, N_RULE_PAD), lambda i: (0, 0)))
    specs.append(pl.BlockSpec((8, N_RULE_PAD), lambda i: (0, 0)))
    return pl.pallas_call(
        _tc_body,
        grid=(_GRID,),
        in_specs=specs,
        out_specs=pl.BlockSpec((1, 1), lambda i: (0, 0)),
        out_shape=jax.ShapeDtypeStruct((1, 1), jnp.float32),
    )(ent_rows, ent_rows, ent_rows, ent_rows, ent_rows, ent_rows,
      ent_rows, ent_rows, ent_rows, ent_rows,
      rel_emb, norm_vec, posr, nrq, nrq, nrq, nrq,
      dr_rows, wr_rows, r1b, confb)


def kernel(pos_triples, neg_triples, ent_emb, rel_emb, norm_vec,
           rule_r1, rule_r2, rule_conf):
    ph, pr, pt = pos_triples[:, 0], pos_triples[:, 1], pos_triples[:, 2]
    nh, nr, nt = neg_triples[:, 0], neg_triples[:, 1], neg_triples[:, 2]

    qmaj = lambda x: x.reshape(POS_B, NEG_RATIO).T.reshape(-1)
    nhq, ntq, nrq = qmaj(nh), qmaj(nt), qmaj(nr)

    eidx = jnp.concatenate([ph, pt, nhq, ntq])
    rulidx = jnp.concatenate(
        [rule_r2, jnp.zeros((N_RULE_PAD - N_RULES,), jnp.int32)])

    ent_rows, dr_rows, wr_rows = _sc_gather(
        ent_emb, rel_emb, norm_vec, eidx, rulidx)

    posr = pr.reshape(POS_B, 1)
    nrq2 = nrq.reshape(NEG_B, 1)
    pad_i = jnp.full((N_RULE_PAD - N_RULES,), -1, jnp.int32)
    r1b = jnp.broadcast_to(
        jnp.concatenate([rule_r1, pad_i])[None, :], (8, N_RULE_PAD))
    confb = jnp.broadcast_to(
        jnp.concatenate([rule_conf, jnp.zeros((N_RULE_PAD - N_RULES,),
                                              jnp.float32)])[None, :],
        (8, N_RULE_PAD))

    loss = _tc_call(ent_rows, rel_emb, norm_vec, posr, nrq2, dr_rows,
                    wr_rows, r1b, confb)
    return loss.reshape(())


# speedup vs baseline: 1.2514x; 1.1613x over previous
import functools

import jax
import jax.numpy as jnp
from jax import lax
from jax.experimental import pallas as pl
from jax.experimental.pallas import tpu as pltpu
from jax.experimental.pallas import tpu_sc as plsc

POS_B = 4096
NEG_B = 16384
DIM = 128
NUM_REL = 1000
NEG_RATIO = NEG_B // POS_B
N_RULES = 20
N_RULE_PAD = 32
MARGIN = 1.0
RULE_WEIGHT = 0.5

ENT_N = 2 * POS_B + 2 * NEG_B

_NW = 32
_CH = 128
_ENT_PW = ENT_N // _NW
_ENT_CH = _ENT_PW // _CH

_NBUF = 6

_PB = 1024
_GRID = POS_B // _PB
_S = POS_B // _PB


_sc_mesh = plsc.VectorSubcoreMesh(core_axis_name="c", subcore_axis_name="s")


@functools.partial(
    pl.kernel,
    mesh=_sc_mesh,
    out_type=(
        jax.ShapeDtypeStruct((ENT_N, DIM), jnp.float32),
        jax.ShapeDtypeStruct((N_RULE_PAD, DIM), jnp.float32),
        jax.ShapeDtypeStruct((N_RULE_PAD, DIM), jnp.float32),
    ),
    scratch_types=[
        pltpu.VMEM((_ENT_PW,), jnp.int32),
        pltpu.VMEM((N_RULE_PAD,), jnp.int32),
        pltpu.VMEM((_NBUF, _CH, DIM), jnp.float32),
        pltpu.VMEM((N_RULE_PAD, DIM), jnp.float32),
        pltpu.SemaphoreType.DMA,
    ] + [pltpu.SemaphoreType.DMA] * (2 * _NBUF),
)
def _sc_gather(ent_hbm, rel_hbm, norm_hbm, eidx_hbm, rulidx_hbm,
               out_ent, out_dr, out_wr,
               idx_e, idx_rul, rows, rows_rul, sem, *ring_sems):
    wid = lax.axis_index("s") * 2 + lax.axis_index("c")
    gsem, ssem = ring_sems[:_NBUF], ring_sems[_NBUF:]

    pltpu.sync_copy(eidx_hbm.at[pl.ds(wid * _ENT_PW, _ENT_PW)], idx_e)

    n = _ENT_CH
    gh = [None] * _NBUF
    sh = [None] * _NBUF
    issued = 0
    for k in range(n):
        while issued < min(n, k + _NBUF):
            b = issued % _NBUF
            if sh[b] is not None:
                sh[b].wait()
            gh[b] = pltpu.async_copy(
                ent_hbm.at[idx_e.at[pl.ds(issued * _CH, _CH)]],
                rows.at[b], gsem[b])
            issued += 1
        b = k % _NBUF
        gh[b].wait()
        sh[b] = pltpu.async_copy(
            rows.at[b], out_ent.at[pl.ds(wid * _ENT_PW + k * _CH, _CH)],
            ssem[b])
    for b in range(min(_NBUF, n)):
        sh[b].wait()

    @pl.when(wid == 0)
    def _():
        pltpu.sync_copy(rulidx_hbm, idx_rul)
        pltpu.async_copy(rel_hbm.at[idx_rul], rows_rul, sem).wait()
        pltpu.sync_copy(rows_rul, out_dr)
        pltpu.async_copy(norm_hbm.at[idx_rul], rows_rul, sem).wait()
        pltpu.sync_copy(rows_rul, out_wr)


def _normw(w):
    return w / (jnp.sqrt(jnp.sum(w * w, axis=-1, keepdims=True)) + 1e-9)


def _tc_body(hp, tp, hn0, hn1, hn2, hn3, tn0, tn1, tn2, tn3,
             catt, posr, nr0, nr1, nr2, nr3,
             dr, wr, r1b, confb, out):
    i = pl.program_id(0)

    cat = catt[...]
    catb = jnp.concatenate(
        [cat[:, :DIM].astype(jnp.bfloat16),
         _normw(cat[:, DIM:]).astype(jnp.bfloat16)], axis=1)
    iot = lax.broadcasted_iota(jnp.int32, (1, NUM_REL), 1)
    dsel_n = (((1,), (0,)), ((), ()))

    def _rel_rows(ridx):
        oh = (ridx[...] == iot).astype(jnp.bfloat16)
        sel = lax.dot_general(oh, catb, dsel_n,
                              preferred_element_type=jnp.float32)
        return sel[:, :DIM], sel[:, DIM:]

    def _score_u(u, d, w):
        al = jnp.sum(w * u, axis=-1, keepdims=True)
        v = u - al * w + d
        return -jnp.sqrt(jnp.sum(v * v, axis=-1, keepdims=True) + 1e-12)

    up = hp[...] - tp[...]
    dp, wp = _rel_rows(posr)
    ps = _score_u(up, dp, wp)

    basic = jnp.float32(0.0)
    for hn, tn, rn in ((hn0, tn0, nr0), (hn1, tn1, nr1),
                       (hn2, tn2, nr2), (hn3, tn3, nr3)):
        dn, wn_ = _rel_rows(rn)
        ns = _score_u(hn[...] - tn[...], dn, wn_)
        basic = basic + jnp.sum(jax.nn.relu(MARGIN - ps + ns))

    drv = dr[...]
    wrv = _normw(wr[...])
    dn_ = (((1,), (1,)), ((), ()))
    alr = lax.dot_general(up, wrv, dn_, preferred_element_type=jnp.float32)
    ber = lax.dot_general(up, drv, dn_, preferred_element_type=jnp.float32)
    ones = jnp.ones((1, DIM), jnp.float32)
    ddr = lax.dot_general(ones, drv * drv, dn_,
                          preferred_element_type=jnp.float32)
    wdr = lax.dot_general(ones, wrv * drv, dn_,
                          preferred_element_type=jnp.float32)
    nu = jnp.sum(up * up, axis=-1, keepdims=True)
    dist2 = nu - alr * alr + ddr + 2.0 * ber - 2.0 * alr * wdr
    rsc = -jnp.sqrt(jnp.maximum(dist2, 0.0) + 1e-12)
    mask = posr[...] == r1b[0:1, :]
    rulep = -jnp.sum(jnp.where(mask, confb[0:1, :] * rsc, 0.0))

    part = basic * (1.0 / NEG_B) + RULE_WEIGHT * rulep

    @pl.when(i == 0)
    def _():
        out[...] = jnp.zeros_like(out)

    out[...] += part


def _tc_call(ent_rows, cat_tab, posr, nrq, dr_rows, wr_rows, r1b, confb):
    ebs = lambda f: pl.BlockSpec((_PB, DIM), f)
    ibs = lambda f: pl.BlockSpec((_PB, 1), f)
    specs = []
    specs.append(ebs(lambda i: (i, 0)))
    specs.append(ebs(lambda i: (i + _S, 0)))
    for q in range(NEG_RATIO):
        specs.append(ebs(lambda i, q=q: (2 * _S + _S * q + i, 0)))
    for q in range(NEG_RATIO):
        specs.append(ebs(lambda i, q=q: (6 * _S + _S * q + i, 0)))
    specs.append(pl.BlockSpec((NUM_REL, 2 * DIM), lambda i: (0, 0)))
    specs.append(ibs(lambda i: (i, 0)))
    for q in range(NEG_RATIO):
        specs.append(ibs(lambda i, q=q: (_S * q + i, 0)))
    specs.append(pl.BlockSpec((N_RULE_PAD, DIM), lambda i: (0, 0)))
    specs.append(pl.BlockSpec((N_RULE_PAD, DIM), lambda i: (0, 0)))
    specs.append(pl.BlockSpec((8, N_RULE_PAD), lambda i: (0, 0)))
    specs.append(pl.BlockSpec((8, N_RULE_PAD), lambda i: (0, 0)))
    return pl.pallas_call(
        _tc_body,
        grid=(_GRID,),
        in_specs=specs,
        out_specs=pl.BlockSpec((1, 1), lambda i: (0, 0)),
        out_shape=jax.ShapeDtypeStruct((1, 1), jnp.float32),
    )(ent_rows, ent_rows, ent_rows, ent_rows, ent_rows, ent_rows,
      ent_rows, ent_rows, ent_rows, ent_rows,
      cat_tab, posr, nrq, nrq, nrq, nrq,
      dr_rows, wr_rows, r1b, confb)


def kernel(pos_triples, neg_triples, ent_emb, rel_emb, norm_vec,
           rule_r1, rule_r2, rule_conf):
    ph, pr, pt = pos_triples[:, 0], pos_triples[:, 1], pos_triples[:, 2]
    nh, nr, nt = neg_triples[:, 0], neg_triples[:, 1], neg_triples[:, 2]

    qmaj = lambda x: x.reshape(POS_B, NEG_RATIO).T.reshape(-1)
    nhq, ntq, nrq = qmaj(nh), qmaj(nt), qmaj(nr)

    eidx = jnp.concatenate([ph, pt, nhq, ntq])
    rulidx = jnp.concatenate(
        [rule_r2, jnp.zeros((N_RULE_PAD - N_RULES,), jnp.int32)])

    ent_rows, dr_rows, wr_rows = _sc_gather(
        ent_emb, rel_emb, norm_vec, eidx, rulidx)

    posr = pr.reshape(POS_B, 1)
    nrq2 = nrq.reshape(NEG_B, 1)
    pad_i = jnp.full((N_RULE_PAD - N_RULES,), -1, jnp.int32)
    r1b = jnp.broadcast_to(
        jnp.concatenate([rule_r1, pad_i])[None, :], (8, N_RULE_PAD))
    confb = jnp.broadcast_to(
        jnp.concatenate([rule_conf, jnp.zeros((N_RULE_PAD - N_RULES,),
                                              jnp.float32)])[None, :],
        (8, N_RULE_PAD))

    cat_tab = jnp.concatenate([rel_emb, norm_vec], axis=1)
    loss = _tc_call(ent_rows, cat_tab, posr, nrq2, dr_rows,
                    wr_rows, r1b, confb)
    return loss.reshape(())
